# baseline (speedup 1.0000x reference)
import jax
import jax.numpy as jnp
from jax import lax
from jax.experimental import pallas as pl
from jax.experimental.pallas import tpu as pltpu
from jax.experimental.pallas import tpu_sc as plsc

_N = 10000
_E = 320000
_F = 128
_WEXT = 144
_NC, _NS, _L = 2, 16, 16
_NW = _NC * _NS
_CH = 48
_NCH = 212
_EPW = _NCH * _CH
_EPAD = _NW * _EPW
_BN = 1000
_NPAD = 10240
_RPS = _NPAD // _NS


def _prep_body(x_ref, w_ref, a_ref, hext_ref, spk_ref):
    x = x_ref[...]
    w = w_ref[...]
    h = jnp.dot(x, w, preferred_element_type=jnp.float32)
    ones = jnp.ones((x.shape[0], 1), jnp.float32)
    pad = jnp.zeros((x.shape[0], _WEXT - _F - 1), jnp.float32)
    hext_ref[...] = jnp.concatenate([h, ones, pad], axis=1)
    a2 = a_ref[...].reshape(2, _F)
    s12 = lax.dot_general(h, a2, (((1,), (1,)), ((), ())),
                          preferred_element_type=jnp.float32)
    u = lax.bitcast_convert_type(s12, jnp.uint32)
    packed = (u[:, 0:1] & jnp.uint32(0xFFFF0000)) | (u[:, 1:2] >> 16)
    spk_ref[...] = lax.bitcast_convert_type(packed, jnp.float32)


def _prep(x, w, a):
    return pl.pallas_call(
        _prep_body,
        grid=(_N // _BN,),
        in_specs=[
            pl.BlockSpec((_BN, _F), lambda i: (i, 0)),
            pl.BlockSpec((_F, _F), lambda i: (0, 0)),
            pl.BlockSpec((1, 2 * _F), lambda i: (0, 0)),
        ],
        out_specs=[
            pl.BlockSpec((_BN, _WEXT), lambda i: (i, 0)),
            pl.BlockSpec((_BN, 1), lambda i: (i, 0)),
        ],
        out_shape=[
            jax.ShapeDtypeStruct((_N, _WEXT), jnp.float32),
            jax.ShapeDtypeStruct((_N, 1), jnp.float32),
        ],
    )(x, w, a)


def _sc_body(src_hbm, dst_hbm, hext_hbm, spk_hbm, part_hbm,
             si0, di0, si1, di1, si2, di2, si3, di3, spk_v,
             r0, r1, r2, r3, acc_sh,
             sg0, sg1, sg2, sg3, ss0, ss1, ss2, ss3,
             sj0, sj1, sj2, sj3):
    cid = lax.axis_index("c")
    sid = lax.axis_index("s")
    wid = sid * _NC + cid
    sidx = (si0, si1, si2, si3)
    didx = (di0, di1, di2, di3)
    rows = (r0, r1, r2, r3)
    semg = (sg0, sg1, sg2, sg3)
    sems = (ss0, ss1, ss2, ss3)
    semi = (sj0, sj1, sj2, sj3)
    mask_hi = jnp.full((_L,), -65536, jnp.int32)

    pltpu.async_copy(spk_hbm, spk_v, sg0).wait()

    @pl.loop(0, 40)
    def _zero_rows(r):
        for j in range(_WEXT // _L):
            r0[r, pl.ds(j * _L, _L)] = jnp.zeros((_L,), jnp.float32)

    rowbase = sid * _RPS
    for z in range(_RPS // 40):
        pltpu.sync_copy(r0.at[pl.ds(0, 40)],
                        acc_sh.at[pl.ds(rowbase + z * 40, 40)])
    plsc.subcore_barrier()

    def issue_idx(k, b):
        pltpu.async_copy(src_hbm.at[wid, k], sidx[b].at[0], semi[b])
        pltpu.async_copy(dst_hbm.at[wid, k], didx[b].at[0], semi[b])

    def wait_idx(k, b):
        pltpu.make_async_copy(src_hbm.at[wid, k], sidx[b].at[0],
                              semi[b]).wait()
        pltpu.make_async_copy(dst_hbm.at[wid, k], didx[b].at[0],
                              semi[b]).wait()

    def issue_gather(k, b):
        pltpu.async_copy(hext_hbm.at[didx[b].at[0]], rows[b], semg[b])

    def wait_gather(k, b):
        pltpu.make_async_copy(hext_hbm.at[didx[b].at[0]], rows[b],
                              semg[b]).wait()

    def issue_scatter(k, b):
        pltpu.async_copy(rows[b], acc_sh.at[sidx[b].at[0]], sems[b],
                         add=True)

    def wait_scatter(k, b):
        pltpu.make_async_copy(rows[b], acc_sh.at[sidx[b].at[0]],
                              sems[b]).wait()

    def compute(k, b):
        row_v = rows[b]
        for g in range(_CH // _L):
            s16 = sidx[b][0, pl.ds(g * _L, _L)]
            d16 = didx[b][0, pl.ds(g * _L, _L)]
            v1 = plsc.load_gather(spk_v, [s16])
            v2 = plsc.load_gather(spk_v, [d16])
            s1 = plsc.bitcast(plsc.bitcast(v1, jnp.int32) & mask_hi,
                              jnp.float32)
            s2 = plsc.bitcast(plsc.bitcast(v2, jnp.int32) << 16, jnp.float32)
            t = s1 + s2
            e16 = jnp.exp(jnp.where(t > 0, -t, -0.2 * t))
            for i in range(_L):
                es = e16[i]
                row = g * _L + i
                for j in range(_WEXT // _L):
                    sl = pl.ds(j * _L, _L)
                    row_v[row, sl] = row_v[row, sl] * es

    pltpu.sync_copy(src_hbm.at[wid, 0], si0.at[0])
    pltpu.sync_copy(dst_hbm.at[wid, 0], di0.at[0])
    pltpu.sync_copy(src_hbm.at[wid, 1], si1.at[0])
    pltpu.sync_copy(dst_hbm.at[wid, 1], di1.at[0])
    issue_gather(0, 0)
    issue_gather(1, 1)
    issue_idx(2, 2)

    @pl.loop(0, _NCH, step=4)
    def _quad(c):
        for j in range(4):
            cj = c + j
            b = j
            bm1 = (j + 3) % 4
            bp2 = (j + 2) % 4
            bp3 = (j + 3) % 4
            wait_gather(cj, b)
            compute(cj, b)
            issue_scatter(cj, b)

            @pl.when(cj >= 1)
            def _(cj=cj, bm1=bm1):
                wait_scatter(cj - 1, bm1)

            @pl.when(cj + 3 < _NCH)
            def _(cj=cj, bp3=bp3):
                issue_idx(cj + 3, bp3)

            @pl.when(cj + 2 < _NCH)
            def _(cj=cj, bp2=bp2):
                wait_idx(cj + 2, bp2)
                issue_gather(cj + 2, bp2)

    wait_scatter(_NCH - 1, 3)

    plsc.subcore_barrier()
    for z in range(_RPS // _CH + 1):
        nr = min(_CH, _RPS - z * _CH)
        r0w = rowbase + z * _CH
        pltpu.sync_copy(acc_sh.at[pl.ds(r0w, nr)],
                        part_hbm.at[cid, pl.ds(r0w, nr)])


def _sc_accumulate(srcd, dstd, hext, spk):
    mesh = plsc.VectorSubcoreMesh(core_axis_name="c", subcore_axis_name="s")
    kern = pl.kernel(
        _sc_body,
        out_type=jax.ShapeDtypeStruct((_NC, _NPAD, _WEXT), jnp.float32),
        mesh=mesh,
        scratch_types=[
            pltpu.VMEM((1, _CH), jnp.int32),
            pltpu.VMEM((1, _CH), jnp.int32),
            pltpu.VMEM((1, _CH), jnp.int32),
            pltpu.VMEM((1, _CH), jnp.int32),
            pltpu.VMEM((1, _CH), jnp.int32),
            pltpu.VMEM((1, _CH), jnp.int32),
            pltpu.VMEM((1, _CH), jnp.int32),
            pltpu.VMEM((1, _CH), jnp.int32),
            pltpu.VMEM((_NPAD,), jnp.float32),
            pltpu.VMEM((_CH, _WEXT), jnp.float32),
            pltpu.VMEM((_CH, _WEXT), jnp.float32),
            pltpu.VMEM((_CH, _WEXT), jnp.float32),
            pltpu.VMEM((_CH, _WEXT), jnp.float32),
            pltpu.VMEM_SHARED((_NPAD, _WEXT), jnp.float32),
            pltpu.SemaphoreType.DMA,
            pltpu.SemaphoreType.DMA,
            pltpu.SemaphoreType.DMA,
            pltpu.SemaphoreType.DMA,
            pltpu.SemaphoreType.DMA,
            pltpu.SemaphoreType.DMA,
            pltpu.SemaphoreType.DMA,
            pltpu.SemaphoreType.DMA,
            pltpu.SemaphoreType.DMA,
            pltpu.SemaphoreType.DMA,
            pltpu.SemaphoreType.DMA,
            pltpu.SemaphoreType.DMA,
        ],
        compiler_params=pltpu.CompilerParams(use_tc_tiling_on_sc=False,
                                             needs_layout_passes=False),
    )
    return kern(srcd, dstd, hext, spk)


def _final_body(part_ref, out_ref):
    p = part_ref[0] + part_ref[1]
    r = p[:, 0:_F] / p[:, _F:_F + 1]
    out_ref[...] = jnp.where(r > 0, r, jnp.exp(jnp.minimum(r, 0.0)) - 1.0)


def _final(part):
    return pl.pallas_call(
        _final_body,
        grid=(_N // _BN,),
        in_specs=[pl.BlockSpec((_NC, _BN, _WEXT), lambda i: (0, i, 0))],
        out_specs=pl.BlockSpec((_BN, _F), lambda i: (i, 0)),
        out_shape=jax.ShapeDtypeStruct((_N, _F), jnp.float32),
    )(part)


def kernel(input, edge, W, a):
    hext, spk = _prep(input, W, a)
    spk = jnp.concatenate(
        [spk.reshape(_N), jnp.zeros((_NPAD - _N,), jnp.float32)])
    srcp = jnp.concatenate(
        [edge[0], jnp.full((_EPAD - _E,), _NPAD - 1, jnp.int32)])
    dstp = jnp.concatenate(
        [edge[1], jnp.zeros((_EPAD - _E,), jnp.int32)])
    srcd = srcp.reshape(_NW, _NCH, _CH)
    dstd = dstp.reshape(_NW, _NCH, _CH)
    part = _sc_accumulate(srcd, dstd, hext, spk)
    return _final(part)
